# baseline (speedup 1.0000x reference)
import numpy as np
import jax
import jax.numpy as jnp
from jax.experimental import pallas as pl
from jax.experimental.pallas import tpu as pltpu

_B, _T, _N, _F = 64, 50, 68, 128
_H = 256
_K = 5
_NCLS = 500
_NP2 = _N + 2 * _K
_TS = 5
_BC = 8
_TR = 10


def _deg_isqrt_np():
    deg = np.array([min(_N - 1, i + _K) - max(0, i - _K) + 1 for i in range(_N)],
                   np.float32)
    return (1.0 / np.sqrt(deg)).astype(np.float32)


def _gcn_body(x_ref, x2_ref, dis_ref, c2_ref, iv_ref, w0, bp0, w1, bp1,
              w2, bp2, out_ref):
    zpad = jnp.zeros((_TS, _K, _BC, _F), jnp.float32)
    dis = dis_ref[...][None, :, None]
    c2 = c2_ref[...][None, :, None]
    iv = iv_ref[...][None, :, None]
    half = _T // (2 * _TS)
    for tc in range(_T // _TS):
        src = x_ref if tc < half else x2_ref
        lo = (tc % half) * _TS
        r = jnp.transpose(src[:, lo:lo + _TS],
                          (1, 2, 0, 3)) * dis
        for w_ref, bp_ref in ((w0, bp0), (w1, bp1), (w2, bp2)):
            u = jnp.reshape(jnp.reshape(r, (_TS * _N * _BC, _F)) @ w_ref[...],
                            (_TS, _N, _BC, _F))
            p = jnp.concatenate([zpad, u, zpad], axis=1)
            p2 = p[:, 0:_NP2 - 1] + p[:, 1:_NP2]
            p4 = p2[:, 0:_NP2 - 3] + p2[:, 2:_NP2 - 1]
            p8 = p4[:, 0:_N] + p4[:, 4:_N + 4]
            win = p8 + p2[:, 8:_N + 8] + p[:, 10:_N + 10]
            r = jnp.maximum(win * c2 + bp_ref[...][None, :, None], 0.0)
        emb = jnp.sum(r * iv, axis=1) * (1.0 / _N)
        out_ref[tc * _TS:(tc + 1) * _TS] = emb


def _gcn(x4d, dis, c2, iv, w0, bp0, w1, bp1, w2, bp2):
    const2 = lambda c: (0, 0)
    return pl.pallas_call(
        _gcn_body,
        grid=(_B // _BC,),
        in_specs=[
            pl.BlockSpec((_BC, _T // 2, _N, _F), lambda c: (c, 0, 0, 0)),
            pl.BlockSpec((_BC, _T // 2, _N, _F), lambda c: (c, 1, 0, 0)),
            pl.BlockSpec((_N, _F), const2),
            pl.BlockSpec((_N, _F), const2),
            pl.BlockSpec((_N, _F), const2),
            pl.BlockSpec((_F, _F), const2), pl.BlockSpec((_N, _F), const2),
            pl.BlockSpec((_F, _F), const2), pl.BlockSpec((_N, _F), const2),
            pl.BlockSpec((_F, _F), const2), pl.BlockSpec((_N, _F), const2),
        ],
        out_specs=pl.BlockSpec((_T, _BC, _F), lambda c: (0, c, 0)),
        out_shape=jax.ShapeDtypeStruct((_T, _B, _F), jnp.float32),
        compiler_params=pltpu.CompilerParams(
            vmem_limit_bytes=128 * 1024 * 1024),
    )(x4d, x4d, dis, c2, iv, w0, bp0, w1, bp1, w2, bp2)


def _lstm_step(x, h_ref, c_ref, w_ref, b_ref):
    g = jnp.concatenate([x, h_ref[...]], axis=1) @ w_ref[...] + b_ref[...]
    i = jax.nn.sigmoid(g[:, 0:_H])
    f = jax.nn.sigmoid(g[:, _H:2 * _H])
    gg = jnp.tanh(g[:, 2 * _H:3 * _H])
    o = jax.nn.sigmoid(g[:, 3 * _H:4 * _H])
    c = f * c_ref[...] + i * gg
    h = o * jnp.tanh(c)
    c_ref[...] = c
    h_ref[...] = h
    return h


def _rec0_body(xf_ref, xb_ref, wf, bf, wb, bb, outf_ref, outb_ref,
               hf, cf, hb, cb):
    s = pl.program_id(0)

    @pl.when(s == 0)
    def _init():
        z = jnp.zeros((_B, _H), jnp.float32)
        hf[...] = z
        cf[...] = z
        hb[...] = z
        cb[...] = z

    for tt in range(_TR):
        outf_ref[tt] = _lstm_step(xf_ref[tt], hf, cf, wf, bf)
        outb_ref[_TR - 1 - tt] = _lstm_step(xb_ref[_TR - 1 - tt], hb, cb,
                                            wb, bb)


def _rec0(emb, wf, bf, wb, bb):
    const2 = lambda s: (0, 0)
    nsteps = _T // _TR
    return pl.pallas_call(
        _rec0_body,
        grid=(nsteps,),
        in_specs=[
            pl.BlockSpec((_TR, _B, _F), lambda s: (s, 0, 0)),
            pl.BlockSpec((_TR, _B, _F), lambda s: (nsteps - 1 - s, 0, 0)),
            pl.BlockSpec((_F + _H, 4 * _H), const2),
            pl.BlockSpec((1, 4 * _H), const2),
            pl.BlockSpec((_F + _H, 4 * _H), const2),
            pl.BlockSpec((1, 4 * _H), const2),
        ],
        out_specs=[
            pl.BlockSpec((_TR, _B, _H), lambda s: (s, 0, 0)),
            pl.BlockSpec((_TR, _B, _H), lambda s: (nsteps - 1 - s, 0, 0)),
        ],
        out_shape=[
            jax.ShapeDtypeStruct((_T, _B, _H), jnp.float32),
            jax.ShapeDtypeStruct((_T, _B, _H), jnp.float32),
        ],
        scratch_shapes=[pltpu.VMEM((_B, _H), jnp.float32)] * 4,
    )(emb, emb, wf, bf, wb, bb)


def _rec1_body(fa_ref, ba_ref, fd_ref, bd_ref, wf, bf, wb, bb,
               w1, b1, w2, b2, out_ref, hf, cf, hb, cb):
    s = pl.program_id(0)

    @pl.when(s == 0)
    def _init():
        z = jnp.zeros((_B, _H), jnp.float32)
        hf[...] = z
        cf[...] = z
        hb[...] = z
        cb[...] = z

    for tt in range(_TR):
        xf = jnp.concatenate([fa_ref[tt], ba_ref[tt]], axis=1)
        hfv = _lstm_step(xf, hf, cf, wf, bf)
        xb = jnp.concatenate([fd_ref[_TR - 1 - tt], bd_ref[_TR - 1 - tt]],
                             axis=1)
        hbv = _lstm_step(xb, hb, cb, wb, bb)

    @pl.when(s == _T // _TR - 1)
    def _cls():
        hcat = jnp.concatenate([hfv, hbv], axis=1)
        hid = jnp.maximum(hcat @ w1[...] + b1[...], 0.0)
        out_ref[...] = hid @ w2[...] + b2[...]


def _rec1(fw0, bw0, wf, bf, wb, bb, w1, b1, w2, b2):
    const2 = lambda s: (0, 0)
    nsteps = _T // _TR
    asc = lambda s: (s, 0, 0)
    dsc = lambda s: (nsteps - 1 - s, 0, 0)
    return pl.pallas_call(
        _rec1_body,
        grid=(nsteps,),
        in_specs=[
            pl.BlockSpec((_TR, _B, _H), asc),
            pl.BlockSpec((_TR, _B, _H), asc),
            pl.BlockSpec((_TR, _B, _H), dsc),
            pl.BlockSpec((_TR, _B, _H), dsc),
            pl.BlockSpec((2 * _H + _H, 4 * _H), const2),
            pl.BlockSpec((1, 4 * _H), const2),
            pl.BlockSpec((2 * _H + _H, 4 * _H), const2),
            pl.BlockSpec((1, 4 * _H), const2),
            pl.BlockSpec((2 * _H, _H), const2),
            pl.BlockSpec((1, _H), const2),
            pl.BlockSpec((_H, _NCLS), const2),
            pl.BlockSpec((1, _NCLS), const2),
        ],
        out_specs=pl.BlockSpec((_B, _NCLS), const2),
        out_shape=jax.ShapeDtypeStruct((_B, _NCLS), jnp.float32),
        scratch_shapes=[pltpu.VMEM((_B, _H), jnp.float32)] * 4,
    )(fw0, bw0, fw0, bw0, wf, bf, wb, bb, w1, b1, w2, b2)


def kernel(x_temporal, gcn_W0, gcn_b0, gcn_W1, gcn_b1, gcn_W2, gcn_b2,
           lstm_fw_Wih0, lstm_fw_Whh0, lstm_fw_b0,
           lstm_bw_Wih0, lstm_bw_Whh0, lstm_bw_b0,
           lstm_fw_Wih1, lstm_fw_Whh1, lstm_fw_b1,
           lstm_bw_Wih1, lstm_bw_Whh1, lstm_bw_b1,
           cls_W1, cls_b1, cls_W2, cls_b2):
    dis_np = _deg_isqrt_np()
    dis = jnp.asarray(np.repeat(dis_np[:, None], _F, axis=1))
    c2 = jnp.asarray(np.repeat((dis_np ** 2)[:, None], _F, axis=1))
    iv = jnp.asarray(np.repeat((1.0 / dis_np)[:, None], _F, axis=1))
    disv = jnp.asarray(dis_np[:, None])
    bp0 = disv * gcn_b0[None, :]
    bp1 = disv * gcn_b1[None, :]
    bp2 = disv * gcn_b2[None, :]

    emb = _gcn(x_temporal, dis, c2, iv, gcn_W0, bp0, gcn_W1, bp1,
               gcn_W2, bp2)

    w0f = jnp.concatenate([lstm_fw_Wih0.T, lstm_fw_Whh0.T], axis=0)
    w0b = jnp.concatenate([lstm_bw_Wih0.T, lstm_bw_Whh0.T], axis=0)
    fw0, bw0 = _rec0(emb, w0f, lstm_fw_b0[None], w0b, lstm_bw_b0[None])

    w1f = jnp.concatenate([lstm_fw_Wih1.T, lstm_fw_Whh1.T], axis=0)
    w1b = jnp.concatenate([lstm_bw_Wih1.T, lstm_bw_Whh1.T], axis=0)
    return _rec1(fw0, bw0, w1f, lstm_fw_b1[None], w1b, lstm_bw_b1[None],
                 cls_W1, cls_b1[None], cls_W2, cls_b2[None])


# speedup vs baseline: 1.0153x; 1.0153x over previous
import numpy as np
import jax
import jax.numpy as jnp
from jax.experimental import pallas as pl
from jax.experimental.pallas import tpu as pltpu

_B, _T, _N, _F = 64, 50, 68, 128
_H = 256
_K = 5
_NCLS = 500
_NP2 = _N + 2 * _K
_TS = 10
_BC = 8
_TR = 10


def _deg_isqrt_np():
    deg = np.array([min(_N - 1, i + _K) - max(0, i - _K) + 1 for i in range(_N)],
                   np.float32)
    return (1.0 / np.sqrt(deg)).astype(np.float32)


def _gcn_body(x_ref, x2_ref, dis_ref, c2_ref, iv_ref, w0, bp0, w1, bp1,
              w2, bp2, out_ref):
    zpad = jnp.zeros((_TS, _K, _BC, _F), jnp.float32)
    dis = dis_ref[...][None, :, None]
    c2 = c2_ref[...][None, :, None]
    iv = iv_ref[...][None, :, None]
    half = _T // (2 * _TS)
    for tc in range(_T // _TS):
        src = x_ref if tc < half else x2_ref
        lo = (tc % half) * _TS
        r = jnp.transpose(src[:, lo:lo + _TS],
                          (1, 2, 0, 3)) * dis
        for w_ref, bp_ref in ((w0, bp0), (w1, bp1), (w2, bp2)):
            u = jnp.reshape(jnp.reshape(r, (_TS * _N * _BC, _F)) @ w_ref[...],
                            (_TS, _N, _BC, _F))
            p = jnp.concatenate([zpad, u, zpad], axis=1)
            p2 = p[:, 0:_NP2 - 1] + p[:, 1:_NP2]
            p4 = p2[:, 0:_NP2 - 3] + p2[:, 2:_NP2 - 1]
            p8 = p4[:, 0:_N] + p4[:, 4:_N + 4]
            win = p8 + p2[:, 8:_N + 8] + p[:, 10:_N + 10]
            r = jnp.maximum(win * c2 + bp_ref[...][None, :, None], 0.0)
        emb = jnp.sum(r * iv, axis=1) * (1.0 / _N)
        out_ref[tc * _TS:(tc + 1) * _TS] = emb


def _gcn(x4d, dis, c2, iv, w0, bp0, w1, bp1, w2, bp2):
    const2 = lambda c: (0, 0)
    return pl.pallas_call(
        _gcn_body,
        grid=(_B // _BC,),
        in_specs=[
            pl.BlockSpec((_BC, _T // 2, _N, _F), lambda c: (c, 0, 0, 0)),
            pl.BlockSpec((_BC, _T // 2, _N, _F), lambda c: (c, 1, 0, 0)),
            pl.BlockSpec((_N, _F), const2),
            pl.BlockSpec((_N, _F), const2),
            pl.BlockSpec((_N, _F), const2),
            pl.BlockSpec((_F, _F), const2), pl.BlockSpec((_N, _F), const2),
            pl.BlockSpec((_F, _F), const2), pl.BlockSpec((_N, _F), const2),
            pl.BlockSpec((_F, _F), const2), pl.BlockSpec((_N, _F), const2),
        ],
        out_specs=pl.BlockSpec((_T, _BC, _F), lambda c: (0, c, 0)),
        out_shape=jax.ShapeDtypeStruct((_T, _B, _F), jnp.float32),
    )(x4d, x4d, dis, c2, iv, w0, bp0, w1, bp1, w2, bp2)


def _lstm_step(x, h_ref, c_ref, w_ref, b_ref):
    g = jnp.concatenate([x, h_ref[...]], axis=1) @ w_ref[...] + b_ref[...]
    i = jax.nn.sigmoid(g[:, 0:_H])
    f = jax.nn.sigmoid(g[:, _H:2 * _H])
    gg = jnp.tanh(g[:, 2 * _H:3 * _H])
    o = jax.nn.sigmoid(g[:, 3 * _H:4 * _H])
    c = f * c_ref[...] + i * gg
    h = o * jnp.tanh(c)
    c_ref[...] = c
    h_ref[...] = h
    return h


def _rec0_body(xf_ref, xb_ref, wf, bf, wb, bb, outf_ref, outb_ref,
               hf, cf, hb, cb):
    s = pl.program_id(0)

    @pl.when(s == 0)
    def _init():
        z = jnp.zeros((_B, _H), jnp.float32)
        hf[...] = z
        cf[...] = z
        hb[...] = z
        cb[...] = z

    for tt in range(_TR):
        outf_ref[tt] = _lstm_step(xf_ref[tt], hf, cf, wf, bf)
        outb_ref[_TR - 1 - tt] = _lstm_step(xb_ref[_TR - 1 - tt], hb, cb,
                                            wb, bb)


def _rec0(emb, wf, bf, wb, bb):
    const2 = lambda s: (0, 0)
    nsteps = _T // _TR
    return pl.pallas_call(
        _rec0_body,
        grid=(nsteps,),
        in_specs=[
            pl.BlockSpec((_TR, _B, _F), lambda s: (s, 0, 0)),
            pl.BlockSpec((_TR, _B, _F), lambda s: (nsteps - 1 - s, 0, 0)),
            pl.BlockSpec((_F + _H, 4 * _H), const2),
            pl.BlockSpec((1, 4 * _H), const2),
            pl.BlockSpec((_F + _H, 4 * _H), const2),
            pl.BlockSpec((1, 4 * _H), const2),
        ],
        out_specs=[
            pl.BlockSpec((_TR, _B, _H), lambda s: (s, 0, 0)),
            pl.BlockSpec((_TR, _B, _H), lambda s: (nsteps - 1 - s, 0, 0)),
        ],
        out_shape=[
            jax.ShapeDtypeStruct((_T, _B, _H), jnp.float32),
            jax.ShapeDtypeStruct((_T, _B, _H), jnp.float32),
        ],
        scratch_shapes=[pltpu.VMEM((_B, _H), jnp.float32)] * 4,
    )(emb, emb, wf, bf, wb, bb)


def _rec1_body(fa_ref, ba_ref, fd_ref, bd_ref, wf, bf, wb, bb,
               w1, b1, w2, b2, out_ref, hf, cf, hb, cb):
    s = pl.program_id(0)

    @pl.when(s == 0)
    def _init():
        z = jnp.zeros((_B, _H), jnp.float32)
        hf[...] = z
        cf[...] = z
        hb[...] = z
        cb[...] = z

    for tt in range(_TR):
        xf = jnp.concatenate([fa_ref[tt], ba_ref[tt]], axis=1)
        hfv = _lstm_step(xf, hf, cf, wf, bf)
        xb = jnp.concatenate([fd_ref[_TR - 1 - tt], bd_ref[_TR - 1 - tt]],
                             axis=1)
        hbv = _lstm_step(xb, hb, cb, wb, bb)

    @pl.when(s == _T // _TR - 1)
    def _cls():
        hcat = jnp.concatenate([hfv, hbv], axis=1)
        hid = jnp.maximum(hcat @ w1[...] + b1[...], 0.0)
        out_ref[...] = hid @ w2[...] + b2[...]


def _rec1(fw0, bw0, wf, bf, wb, bb, w1, b1, w2, b2):
    const2 = lambda s: (0, 0)
    nsteps = _T // _TR
    asc = lambda s: (s, 0, 0)
    dsc = lambda s: (nsteps - 1 - s, 0, 0)
    return pl.pallas_call(
        _rec1_body,
        grid=(nsteps,),
        in_specs=[
            pl.BlockSpec((_TR, _B, _H), asc),
            pl.BlockSpec((_TR, _B, _H), asc),
            pl.BlockSpec((_TR, _B, _H), dsc),
            pl.BlockSpec((_TR, _B, _H), dsc),
            pl.BlockSpec((2 * _H + _H, 4 * _H), const2),
            pl.BlockSpec((1, 4 * _H), const2),
            pl.BlockSpec((2 * _H + _H, 4 * _H), const2),
            pl.BlockSpec((1, 4 * _H), const2),
            pl.BlockSpec((2 * _H, _H), const2),
            pl.BlockSpec((1, _H), const2),
            pl.BlockSpec((_H, _NCLS), const2),
            pl.BlockSpec((1, _NCLS), const2),
        ],
        out_specs=pl.BlockSpec((_B, _NCLS), const2),
        out_shape=jax.ShapeDtypeStruct((_B, _NCLS), jnp.float32),
        scratch_shapes=[pltpu.VMEM((_B, _H), jnp.float32)] * 4,
    )(fw0, bw0, fw0, bw0, wf, bf, wb, bb, w1, b1, w2, b2)


def kernel(x_temporal, gcn_W0, gcn_b0, gcn_W1, gcn_b1, gcn_W2, gcn_b2,
           lstm_fw_Wih0, lstm_fw_Whh0, lstm_fw_b0,
           lstm_bw_Wih0, lstm_bw_Whh0, lstm_bw_b0,
           lstm_fw_Wih1, lstm_fw_Whh1, lstm_fw_b1,
           lstm_bw_Wih1, lstm_bw_Whh1, lstm_bw_b1,
           cls_W1, cls_b1, cls_W2, cls_b2):
    dis_np = _deg_isqrt_np()
    dis = jnp.asarray(np.repeat(dis_np[:, None], _F, axis=1))
    c2 = jnp.asarray(np.repeat((dis_np ** 2)[:, None], _F, axis=1))
    iv = jnp.asarray(np.repeat((1.0 / dis_np)[:, None], _F, axis=1))
    disv = jnp.asarray(dis_np[:, None])
    bp0 = disv * gcn_b0[None, :]
    bp1 = disv * gcn_b1[None, :]
    bp2 = disv * gcn_b2[None, :]

    emb = _gcn(x_temporal, dis, c2, iv, gcn_W0, bp0, gcn_W1, bp1,
               gcn_W2, bp2)

    w0f = jnp.concatenate([lstm_fw_Wih0.T, lstm_fw_Whh0.T], axis=0)
    w0b = jnp.concatenate([lstm_bw_Wih0.T, lstm_bw_Whh0.T], axis=0)
    fw0, bw0 = _rec0(emb, w0f, lstm_fw_b0[None], w0b, lstm_bw_b0[None])

    w1f = jnp.concatenate([lstm_fw_Wih1.T, lstm_fw_Whh1.T], axis=0)
    w1b = jnp.concatenate([lstm_bw_Wih1.T, lstm_bw_Whh1.T], axis=0)
    return _rec1(fw0, bw0, w1f, lstm_fw_b1[None], w1b, lstm_bw_b1[None],
                 cls_W1, cls_b1[None], cls_W2, cls_b2[None])
